# Initial kernel scaffold; baseline (speedup 1.0000x reference)
import jax
import jax.numpy as jnp
from jax.experimental import pallas as pl


def kernel(x, edge_index, batchs, energy, conv1_W, conv1_asrc, conv1_adst, conv1_b, convs_W, convs_asrc, convs_adst, convs_b, lin0_W, lin0_b, lin1_W, lin1_b, lin2_W, lin2_b, lin3_W, lin3_b):
    raise NotImplementedError("write your pallas kernel here")


# speedup vs baseline: 13.5937x; 13.5937x over previous
import functools

import jax
import jax.numpy as jnp
from jax import lax
from jax.experimental import pallas as pl
from jax.experimental.pallas import tpu as pltpu
from jax.experimental.pallas import tpu_sc as plsc

N = 10000
NP = 10240
E = 320000
NC = 2
NS = 16
NW = NC * NS
EW = 10240
EP = NW * EW
CH = 80
K = 128
D = 128
B = 64
SENT = 10000
ROWS_PER_TILE = NP // NS


def _tc_layer_body(first, t_or_p_ref, s_ref, b_ref, W_ref, asrc_ref, adst_ref,
                   h_ref, ss_ref):
    if first:
        t = t_or_p_ref[...]
    else:
        agg = t_or_p_ref[0] + t_or_p_ref[1]
        s = jnp.sum(s_ref[...], axis=0)
        t = jnp.maximum(agg / (s[:, None] + 1e-16) + b_ref[0][None, :], 0.0)
    h = jnp.dot(t, W_ref[...], preferred_element_type=jnp.float32)
    h_ref[...] = h
    ssrc = jnp.sum(h * asrc_ref[0][None, :], axis=1)
    sdst = jnp.sum(h * adst_ref[0][None, :], axis=1)
    blk = ssrc.shape[0]
    ss_ref[...] = jnp.concatenate(
        [ssrc[None, :], sdst[None, :], jnp.zeros((6, blk), jnp.float32)], axis=0)


def _tc_layer(first, t_or_p, s_parts, b_prev, W, a_src, a_dst):
    BLK = 1024
    grid = (NP // BLK,)
    if first:
        in_specs = [
            pl.BlockSpec((BLK, D), lambda i: (i, 0)),
            pl.BlockSpec((NW, BLK), lambda i: (0, i)),
            pl.BlockSpec((1, D), lambda i: (0, 0)),
            pl.BlockSpec((D, D), lambda i: (0, 0)),
            pl.BlockSpec((1, D), lambda i: (0, 0)),
            pl.BlockSpec((1, D), lambda i: (0, 0)),
        ]
    else:
        in_specs = [
            pl.BlockSpec((2, BLK, D), lambda i: (0, i, 0)),
            pl.BlockSpec((NW, BLK), lambda i: (0, i)),
            pl.BlockSpec((1, D), lambda i: (0, 0)),
            pl.BlockSpec((D, D), lambda i: (0, 0)),
            pl.BlockSpec((1, D), lambda i: (0, 0)),
            pl.BlockSpec((1, D), lambda i: (0, 0)),
        ]
    out_specs = [
        pl.BlockSpec((BLK, D), lambda i: (i, 0)),
        pl.BlockSpec((8, BLK), lambda i: (0, i)),
    ]
    return pl.pallas_call(
        functools.partial(_tc_layer_body, first),
        grid=grid,
        in_specs=in_specs,
        out_specs=out_specs,
        out_shape=[
            jax.ShapeDtypeStruct((NP, D), jnp.float32),
            jax.ShapeDtypeStruct((8, NP), jnp.float32),
        ],
    )(t_or_p, s_parts, b_prev, W, a_src, a_dst)


NT = 10112
SC_CH = 8


def _sc_edge_body(h_hbm, ssrc_h, sdst_h, src_hbm, dst_hbm, z_hbm,
                  outp_hbm, sp_hbm,
                  ssrc_t, sdst_t, src_t, dst_t, e_buf, rows, s_loc,
                  out_acc, sem, sem2):
    cid = lax.axis_index("c")
    sid = lax.axis_index("s")
    wid = cid * NS + sid

    pltpu.sync_copy(ssrc_h.at[pl.ds(0, NT)], ssrc_t)
    pltpu.sync_copy(sdst_h.at[pl.ds(0, NT)], sdst_t)

    ssrc_t[pl.ds(SENT, 16)] = jnp.full((16,), -1e30, jnp.float32)

    def _zs(i, _):
        s_loc[lax.shift_right_logical(i, 3),
              pl.ds(jnp.bitwise_and(i, 7) * 16, 16)] = jnp.zeros(
                  (16,), jnp.float32)
        return 0
    lax.fori_loop(0, NP // 16, _zs, 0)

    pltpu.sync_copy(z_hbm.at[pl.ds(sid * ROWS_PER_TILE, ROWS_PER_TILE)],
                    out_acc.at[pl.ds(sid * ROWS_PER_TILE, ROWS_PER_TILE)])
    plsc.subcore_barrier()

    def _super(sc, _):
        pltpu.sync_copy(src_hbm.at[wid, pl.ds(sc * SC_CH, SC_CH)], src_t)
        pltpu.sync_copy(dst_hbm.at[wid, pl.ds(sc * SC_CH, SC_CH)], dst_t)

        def _chunk(cc, _):
            pltpu.async_copy(h_hbm.at[src_t.at[cc]], rows, sem).wait()

            for g in range(K // 16):
                sv = src_t[cc, pl.ds(g * 16, 16)]
                dv = dst_t[cc, pl.ds(g * 16, 16)]
                a1 = plsc.load_gather(ssrc_t, [sv])
                a2 = plsc.load_gather(sdst_t, [dv])
                al = a1 + a2
                al = jnp.where(al >= 0.0, al, al * 0.2)
                ev = jnp.exp(al)
                e_buf[pl.ds(g * 16, 16)] = ev
                plsc.addupdate_scatter(
                    s_loc, [lax.shift_right_logical(dv, 7),
                            jnp.bitwise_and(dv, 127)], ev)

            def _scale(j, _):
                es = plsc.load_gather(e_buf, [jnp.full((16,), j, jnp.int32)])
                for r in range(D // 16):
                    rows[j, pl.ds(r * 16, 16)] = rows[j, pl.ds(r * 16, 16)] * es
                return 0
            lax.fori_loop(0, K, _scale, 0)

            pltpu.async_copy(rows, out_acc.at[dst_t.at[cc]], sem2,
                             add=True).wait()
            return 0

        lax.fori_loop(0, SC_CH, _chunk, 0)
        return 0

    lax.fori_loop(0, CH // SC_CH, _super, 0)
    plsc.subcore_barrier()

    pltpu.sync_copy(out_acc.at[pl.ds(sid * ROWS_PER_TILE, ROWS_PER_TILE)],
                    outp_hbm.at[cid, pl.ds(sid * ROWS_PER_TILE, ROWS_PER_TILE)])
    pltpu.sync_copy(s_loc, sp_hbm.at[wid])


def _sc_edges(h, ssrc_h, sdst_h, src_r, dst_r, zeros_hbm):
    mesh = plsc.VectorSubcoreMesh(core_axis_name="c", subcore_axis_name="s")
    f = pl.kernel(
        _sc_edge_body,
        out_type=[
            jax.ShapeDtypeStruct((NC, NP, D), jnp.float32),
            jax.ShapeDtypeStruct((NW, NP // K, K), jnp.float32),
        ],
        mesh=mesh,
        compiler_params=pltpu.CompilerParams(needs_layout_passes=False),
        scratch_types=[
            pltpu.VMEM((NT,), jnp.float32),
            pltpu.VMEM((NT,), jnp.float32),
            pltpu.VMEM((SC_CH, K), jnp.int32),
            pltpu.VMEM((SC_CH, K), jnp.int32),
            pltpu.VMEM((K,), jnp.float32),
            pltpu.VMEM((K, D), jnp.float32),
            pltpu.VMEM((NP // K, K), jnp.float32),
            pltpu.VMEM_SHARED((NP, D), jnp.float32),
            pltpu.SemaphoreType.DMA,
            pltpu.SemaphoreType.DMA,
        ],
    )
    return f(h, ssrc_h, sdst_h, src_r, dst_r, zeros_hbm)


def _tc_head_body(p_ref, s_ref, b4_ref, batch_ref, energy_ref,
                  l0w_ref, l0b_ref, l1wa_ref, l1wb_ref, l1b_ref,
                  l2w_ref, l2b_ref, l3w_ref, l3b_ref, out_ref):
    agg = p_ref[0] + p_ref[1]
    s = jnp.sum(s_ref[...], axis=0)
    t = jnp.maximum(agg / (s[:, None] + 1e-16) + b4_ref[0][None, :], 0.0)

    bt = batch_ref[...].reshape(1, NP)
    classes = lax.broadcasted_iota(jnp.int32, (B, NP), 0)
    oh = (bt == classes).astype(jnp.float32)
    g = jnp.dot(oh, t, preferred_element_type=jnp.float32,
                precision=lax.Precision.HIGHEST)

    y = jnp.maximum(
        jnp.dot(g, l0w_ref[...], preferred_element_type=jnp.float32)
        + l0b_ref[0][None, :], 0.0)
    z = jnp.maximum(
        jnp.dot(y, l1wa_ref[...], preferred_element_type=jnp.float32)
        + energy_ref[...] * l1wb_ref[0][None, :] + l1b_ref[0][None, :], 0.0)
    z = z + y
    z = jnp.maximum(
        jnp.dot(z, l2w_ref[...], preferred_element_type=jnp.float32)
        + l2b_ref[0][None, :], 0.0)
    o = (jnp.dot(z, l3w_ref[...], preferred_element_type=jnp.float32)
         + l3b_ref[0][None, :])
    m = jnp.max(o, axis=1, keepdims=True)
    lse = m + jnp.log(jnp.sum(jnp.exp(o - m), axis=1, keepdims=True))
    out_ref[...] = o - lse


def _tc_head(p, s_parts, b4, batchs2d, energy,
             l0w, l0b, l1wa, l1wb, l1b, l2w, l2b, l3w, l3b):
    return pl.pallas_call(
        _tc_head_body,
        out_shape=jax.ShapeDtypeStruct((B, 2), jnp.float32),
    )(p, s_parts, b4, batchs2d, energy,
      l0w, l0b, l1wa, l1wb, l1b, l2w, l2b, l3w, l3b)


def kernel(x, edge_index, batchs, energy, conv1_W, conv1_asrc, conv1_adst,
           conv1_b, convs_W, convs_asrc, convs_adst, convs_b, lin0_W, lin0_b,
           lin1_W, lin1_b, lin2_W, lin2_b, lin3_W, lin3_b):
    f32 = jnp.float32
    x_pad = jnp.zeros((NP, D), f32).at[:N].set(x)
    src = jnp.concatenate(
        [edge_index[0], jnp.full((EP - E,), SENT, jnp.int32)]).reshape(NW, CH, K)
    dst = jnp.concatenate(
        [edge_index[1], jnp.zeros((EP - E,), jnp.int32)]).reshape(NW, CH, K)
    batchs2d = jnp.concatenate(
        [batchs, jnp.full((NP - N,), B, jnp.int32)]).reshape(NP // K, K)
    zeros_hbm = jnp.zeros((NP, D), f32)

    r2 = lambda v: v.reshape(1, D)

    h, ss = _tc_layer(True, x_pad, jnp.zeros((NW, NP), f32), r2(conv1_b),
                      conv1_W, r2(conv1_asrc), r2(conv1_adst))
    p, sp3 = _sc_edges(h, ss[0], ss[1], src, dst, zeros_hbm)
    sp = sp3.reshape(NW, NP)
    bias_prev = conv1_b

    for i in range(3):
        h, ss = _tc_layer(False, p, sp, r2(bias_prev), convs_W[i],
                          r2(convs_asrc[i]), r2(convs_adst[i]))
        p, sp3 = _sc_edges(h, ss[0], ss[1], src, dst, zeros_hbm)
        sp = sp3.reshape(NW, NP)
        bias_prev = convs_b[i]

    return _tc_head(p, sp, r2(bias_prev), batchs2d, energy,
                    lin0_W, lin0_b.reshape(1, D),
                    lin1_W[:D], lin1_W[D:D + 1], lin1_b.reshape(1, D),
                    lin2_W, lin2_b.reshape(1, D),
                    lin3_W, lin3_b.reshape(1, 2))


# speedup vs baseline: 18.2293x; 1.3410x over previous
import functools

import jax
import jax.numpy as jnp
from jax import lax
from jax.experimental import pallas as pl
from jax.experimental.pallas import tpu as pltpu
from jax.experimental.pallas import tpu_sc as plsc

N = 10000
NP = 10240
E = 320000
NC = 2
NS = 16
NW = NC * NS
EW = 10240
EP = NW * EW
CH = 80
K = 128
D = 128
B = 64
SENT = 10000
ROWS_PER_TILE = NP // NS


def _tc_layer_body(first, t_or_p_ref, s_ref, b_ref, W_ref, asrc_ref, adst_ref,
                   h_ref, ss_ref):
    if first:
        t = t_or_p_ref[...]
    else:
        agg = t_or_p_ref[0] + t_or_p_ref[1]
        s = jnp.sum(s_ref[...], axis=0)
        t = jnp.maximum(agg / (s[:, None] + 1e-16) + b_ref[0][None, :], 0.0)
    h = jnp.dot(t, W_ref[...], preferred_element_type=jnp.float32)
    h_ref[...] = h
    ssrc = jnp.sum(h * asrc_ref[0][None, :], axis=1)
    sdst = jnp.sum(h * adst_ref[0][None, :], axis=1)
    blk = ssrc.shape[0]
    pos = lax.broadcasted_iota(jnp.int32, (1, blk), 1)
    sent_local = SENT - pl.program_id(0) * blk
    ssrc2 = jnp.where(pos == sent_local, -1e30, ssrc[None, :])
    ss_ref[...] = jnp.concatenate(
        [ssrc2, sdst[None, :], jnp.zeros((6, blk), jnp.float32)], axis=0)


def _tc_layer(first, t_or_p, s_parts, b_prev, W, a_src, a_dst):
    BLK = 1024
    grid = (NP // BLK,)
    if first:
        in_specs = [
            pl.BlockSpec((BLK, D), lambda i: (i, 0)),
            pl.BlockSpec((NW, BLK), lambda i: (0, i)),
            pl.BlockSpec((1, D), lambda i: (0, 0)),
            pl.BlockSpec((D, D), lambda i: (0, 0)),
            pl.BlockSpec((1, D), lambda i: (0, 0)),
            pl.BlockSpec((1, D), lambda i: (0, 0)),
        ]
    else:
        in_specs = [
            pl.BlockSpec((2, BLK, D), lambda i: (0, i, 0)),
            pl.BlockSpec((NW, BLK), lambda i: (0, i)),
            pl.BlockSpec((1, D), lambda i: (0, 0)),
            pl.BlockSpec((D, D), lambda i: (0, 0)),
            pl.BlockSpec((1, D), lambda i: (0, 0)),
            pl.BlockSpec((1, D), lambda i: (0, 0)),
        ]
    out_specs = [
        pl.BlockSpec((BLK, D), lambda i: (i, 0)),
        pl.BlockSpec((8, BLK), lambda i: (0, i)),
    ]
    return pl.pallas_call(
        functools.partial(_tc_layer_body, first),
        grid=grid,
        in_specs=in_specs,
        out_specs=out_specs,
        out_shape=[
            jax.ShapeDtypeStruct((NP, D), jnp.float32),
            jax.ShapeDtypeStruct((8, NP), jnp.float32),
        ],
    )(t_or_p, s_parts, b_prev, W, a_src, a_dst)


NSUP = 10
SC_CH = 8
NCH = CH


def _sc_edge_body(h_hbm, ssrc_h, sdst_h, src_hbm, dst_hbm, z_hbm,
                  outp_hbm, sp_hbm,
                  src_t, dst_t, dv_buf, e_buf, ssv, sdv, rows, s_loc,
                  out_acc,
                  gs0, gs1, as0, as1, ad0, ad1, ss0, ss1, xs, xd):
    cid = lax.axis_index("c")
    sid = lax.axis_index("s")
    wid = cid * NS + sid
    gs = (gs0, gs1)
    asm = (as0, as1)
    adm = (ad0, ad1)
    ssm = (ss0, ss1)
    i32 = jnp.int32

    def _zs(i, _):
        s_loc[lax.shift_right_logical(i, 3),
              pl.ds(jnp.bitwise_and(i, 7) * 16, 16)] = jnp.zeros(
                  (16,), jnp.float32)
        return 0
    lax.fori_loop(0, NP // 16, _zs, 0)

    pltpu.sync_copy(z_hbm.at[pl.ds(sid * ROWS_PER_TILE, ROWS_PER_TILE)],
                    out_acc.at[pl.ds(sid * ROWS_PER_TILE, ROWS_PER_TILE)])
    plsc.subcore_barrier()

    pltpu.sync_copy(src_hbm.at[wid, pl.ds(0, SC_CH)], src_t.at[0])
    pltpu.sync_copy(dst_hbm.at[wid, pl.ds(0, SC_CH)], dst_t.at[0])
    pltpu.async_copy(ssrc_h.at[src_t.at[0, 0]], ssv.at[0], as0)
    pltpu.async_copy(sdst_h.at[dst_t.at[0, 0]], sdv.at[0], ad0)
    pltpu.async_copy(h_hbm.at[src_t.at[0, 0]], rows.at[0], gs0)

    def _wait(src_ref, dst_ref, sem):
        pltpu.make_async_copy(src_ref, dst_ref, sem).wait()

    def _iter(g, b):
        sup = lax.shift_right_logical(g, 3)
        slot = jnp.bitwise_and(sup, 1)
        row = jnp.bitwise_and(g, 7)

        @pl.when(jnp.logical_and(row == 0, sup + 1 < NSUP))
        def _():
            nslot = jnp.bitwise_and(sup + 1, 1)
            pltpu.async_copy(src_hbm.at[wid, pl.ds((sup + 1) * SC_CH, SC_CH)],
                             src_t.at[nslot], xs)
            pltpu.async_copy(dst_hbm.at[wid, pl.ds((sup + 1) * SC_CH, SC_CH)],
                             dst_t.at[nslot], xd)

        _wait(ssrc_h.at[pl.ds(0, K)], ssv.at[b], asm[b])
        _wait(sdst_h.at[pl.ds(0, K)], sdv.at[b], adm[b])
        for gi in range(K // 16):
            a1 = ssv[b, pl.ds(gi * 16, 16)]
            a2 = sdv[b, pl.ds(gi * 16, 16)]
            al = a1 + a2
            al = jnp.where(al >= 0.0, al, al * 0.2)
            ev = jnp.exp(al)
            e_buf[pl.ds(gi * 16, 16)] = ev
            dv = dst_t[slot, row, pl.ds(gi * 16, 16)]
            plsc.addupdate_scatter(
                s_loc, [lax.shift_right_logical(dv, 7),
                        jnp.bitwise_and(dv, 127)], ev)
            dv_buf[b, pl.ds(gi * 16, 16)] = dv

        b2 = 1 - b
        @pl.when(g + 1 < NCH)
        def _():
            g2 = g + 1
            sup2 = lax.shift_right_logical(g2, 3)
            slot2 = jnp.bitwise_and(sup2, 1)
            row2 = jnp.bitwise_and(g2, 7)

            @pl.when(row2 == 0)
            def _():
                _wait(src_hbm.at[wid, pl.ds(0, SC_CH)], src_t.at[slot2], xs)
                _wait(dst_hbm.at[wid, pl.ds(0, SC_CH)], dst_t.at[slot2], xd)

            pltpu.async_copy(ssrc_h.at[src_t.at[slot2, row2]], ssv.at[b2],
                             asm[b2])
            pltpu.async_copy(sdst_h.at[dst_t.at[slot2, row2]], sdv.at[b2],
                             adm[b2])

            @pl.when(g >= 1)
            def _():
                _wait(rows.at[b2], out_acc.at[pl.ds(0, K)], ssm[b2])

            pltpu.async_copy(h_hbm.at[src_t.at[slot2, row2]], rows.at[b2],
                             gs[b2])

        _wait(h_hbm.at[pl.ds(0, K)], rows.at[b], gs[b])

        def _scale(j2, _):
            j = j2 * 2
            for dj in range(2):
                es = plsc.load_gather(
                    e_buf, [jnp.full((16,), j + dj, i32)])
                for r in range(D // 16):
                    rows[b, j + dj, pl.ds(r * 16, 16)] = (
                        rows[b, j + dj, pl.ds(r * 16, 16)] * es)
            return 0
        lax.fori_loop(0, K // 2, _scale, 0)

        pltpu.async_copy(rows.at[b], out_acc.at[dv_buf.at[b]], ssm[b],
                         add=True)

    def _pair(i, _):
        _iter(i * 2, 0)
        _iter(i * 2 + 1, 1)
        return 0
    lax.fori_loop(0, NCH // 2, _pair, 0)

    pltpu.make_async_copy(rows.at[0], out_acc.at[pl.ds(0, K)], ss0).wait()
    pltpu.make_async_copy(rows.at[1], out_acc.at[pl.ds(0, K)], ss1).wait()
    plsc.subcore_barrier()

    pltpu.sync_copy(out_acc.at[pl.ds(sid * ROWS_PER_TILE, ROWS_PER_TILE)],
                    outp_hbm.at[cid, pl.ds(sid * ROWS_PER_TILE, ROWS_PER_TILE)])
    pltpu.sync_copy(s_loc, sp_hbm.at[wid])


def _sc_edges(h, ssrc_h, sdst_h, src_r, dst_r, zeros_hbm):
    mesh = plsc.VectorSubcoreMesh(core_axis_name="c", subcore_axis_name="s")
    f = pl.kernel(
        _sc_edge_body,
        out_type=[
            jax.ShapeDtypeStruct((NC, NP, D), jnp.float32),
            jax.ShapeDtypeStruct((NW, NP // K, K), jnp.float32),
        ],
        mesh=mesh,
        compiler_params=pltpu.CompilerParams(needs_layout_passes=False),
        scratch_types=[
            pltpu.VMEM((2, SC_CH, K), jnp.int32),
            pltpu.VMEM((2, SC_CH, K), jnp.int32),
            pltpu.VMEM((2, K), jnp.int32),
            pltpu.VMEM((K,), jnp.float32),
            pltpu.VMEM((2, K), jnp.float32),
            pltpu.VMEM((2, K), jnp.float32),
            pltpu.VMEM((2, K, D), jnp.float32),
            pltpu.VMEM((NP // K, K), jnp.float32),
            pltpu.VMEM_SHARED((NP, D), jnp.float32),
            pltpu.SemaphoreType.DMA,
            pltpu.SemaphoreType.DMA,
            pltpu.SemaphoreType.DMA,
            pltpu.SemaphoreType.DMA,
            pltpu.SemaphoreType.DMA,
            pltpu.SemaphoreType.DMA,
            pltpu.SemaphoreType.DMA,
            pltpu.SemaphoreType.DMA,
            pltpu.SemaphoreType.DMA,
            pltpu.SemaphoreType.DMA,
        ],
    )
    return f(h, ssrc_h, sdst_h, src_r, dst_r, zeros_hbm)


def _tc_head_body(p_ref, s_ref, b4_ref, batch_ref, energy_ref,
                  l0w_ref, l0b_ref, l1wa_ref, l1wb_ref, l1b_ref,
                  l2w_ref, l2b_ref, l3w_ref, l3b_ref, out_ref):
    agg = p_ref[0] + p_ref[1]
    s = jnp.sum(s_ref[...], axis=0)
    t = jnp.maximum(agg / (s[:, None] + 1e-16) + b4_ref[0][None, :], 0.0)

    bt = batch_ref[...].reshape(1, NP)
    classes = lax.broadcasted_iota(jnp.int32, (B, NP), 0)
    oh = (bt == classes).astype(jnp.float32)
    g = jnp.dot(oh, t, preferred_element_type=jnp.float32,
                precision=lax.Precision.HIGHEST)

    y = jnp.maximum(
        jnp.dot(g, l0w_ref[...], preferred_element_type=jnp.float32)
        + l0b_ref[0][None, :], 0.0)
    z = jnp.maximum(
        jnp.dot(y, l1wa_ref[...], preferred_element_type=jnp.float32)
        + energy_ref[...] * l1wb_ref[0][None, :] + l1b_ref[0][None, :], 0.0)
    z = z + y
    z = jnp.maximum(
        jnp.dot(z, l2w_ref[...], preferred_element_type=jnp.float32)
        + l2b_ref[0][None, :], 0.0)
    o = (jnp.dot(z, l3w_ref[...], preferred_element_type=jnp.float32)
         + l3b_ref[0][None, :])
    m = jnp.max(o, axis=1, keepdims=True)
    lse = m + jnp.log(jnp.sum(jnp.exp(o - m), axis=1, keepdims=True))
    out_ref[...] = o - lse


def _tc_head(p, s_parts, b4, batchs2d, energy,
             l0w, l0b, l1wa, l1wb, l1b, l2w, l2b, l3w, l3b):
    return pl.pallas_call(
        _tc_head_body,
        out_shape=jax.ShapeDtypeStruct((B, 2), jnp.float32),
    )(p, s_parts, b4, batchs2d, energy,
      l0w, l0b, l1wa, l1wb, l1b, l2w, l2b, l3w, l3b)


def kernel(x, edge_index, batchs, energy, conv1_W, conv1_asrc, conv1_adst,
           conv1_b, convs_W, convs_asrc, convs_adst, convs_b, lin0_W, lin0_b,
           lin1_W, lin1_b, lin2_W, lin2_b, lin3_W, lin3_b):
    f32 = jnp.float32
    x_pad = jnp.zeros((NP, D), f32).at[:N].set(x)
    src = jnp.concatenate(
        [edge_index[0], jnp.full((EP - E,), SENT, jnp.int32)]).reshape(NW, CH, K)
    dst = jnp.concatenate(
        [edge_index[1], jnp.zeros((EP - E,), jnp.int32)]).reshape(NW, CH, K)
    batchs2d = jnp.concatenate(
        [batchs, jnp.full((NP - N,), B, jnp.int32)]).reshape(NP // K, K)
    zeros_hbm = jnp.zeros((NP, D), f32)

    r2 = lambda v: v.reshape(1, D)

    h, ss = _tc_layer(True, x_pad, jnp.zeros((NW, NP), f32), r2(conv1_b),
                      conv1_W, r2(conv1_asrc), r2(conv1_adst))
    p, sp3 = _sc_edges(h, ss[0], ss[1], src, dst, zeros_hbm)
    sp = sp3.reshape(NW, NP)
    bias_prev = conv1_b

    for i in range(3):
        h, ss = _tc_layer(False, p, sp, r2(bias_prev), convs_W[i],
                          r2(convs_asrc[i]), r2(convs_adst[i]))
        p, sp3 = _sc_edges(h, ss[0], ss[1], src, dst, zeros_hbm)
        sp = sp3.reshape(NW, NP)
        bias_prev = convs_b[i]

    return _tc_head(p, sp, r2(bias_prev), batchs2d, energy,
                    lin0_W, lin0_b.reshape(1, D),
                    lin1_W[:D], lin1_W[D:D + 1], lin1_b.reshape(1, D),
                    lin2_W, lin2_b.reshape(1, D),
                    lin3_W, lin3_b.reshape(1, 2))


# speedup vs baseline: 21.5544x; 1.1824x over previous
import functools

import jax
import jax.numpy as jnp
from jax import lax
from jax.experimental import pallas as pl
from jax.experimental.pallas import tpu as pltpu
from jax.experimental.pallas import tpu_sc as plsc

N = 10000
NP = 10240
E = 320000
NC = 2
NS = 16
NW = NC * NS
EW = 10240
EP = NW * EW
CH = 320
K = 32
D = 128
B = 64
SENT = 10000
ROWS_PER_TILE = NP // NS


def _tc_layer_body(first, t_or_p_ref, s_ref, b_ref, W_ref, asrc_ref, adst_ref,
                   h_ref, ss_ref):
    if first:
        t = t_or_p_ref[...]
    else:
        agg = t_or_p_ref[0] + t_or_p_ref[1]
        s = jnp.sum(s_ref[...], axis=0)
        t = jnp.maximum(agg / (s[:, None] + 1e-16) + b_ref[0][None, :], 0.0)
    h = jnp.dot(t, W_ref[...], preferred_element_type=jnp.float32)
    h_ref[...] = h
    ssrc = jnp.sum(h * asrc_ref[0][None, :], axis=1)
    sdst = jnp.sum(h * adst_ref[0][None, :], axis=1)
    blk = ssrc.shape[0]
    pos = lax.broadcasted_iota(jnp.int32, (1, blk), 1)
    sent_local = SENT - pl.program_id(0) * blk
    ssrc2 = jnp.where(pos == sent_local, -1e30, ssrc[None, :])
    ss_ref[...] = jnp.concatenate(
        [ssrc2, sdst[None, :], jnp.zeros((6, blk), jnp.float32)], axis=0)


def _tc_layer(first, t_or_p, s_parts, b_prev, W, a_src, a_dst):
    BLK = 1024
    grid = (NP // BLK,)
    if first:
        in_specs = [
            pl.BlockSpec((BLK, D), lambda i: (i, 0)),
            pl.BlockSpec((NW, BLK), lambda i: (0, i)),
            pl.BlockSpec((1, D), lambda i: (0, 0)),
            pl.BlockSpec((D, D), lambda i: (0, 0)),
            pl.BlockSpec((1, D), lambda i: (0, 0)),
            pl.BlockSpec((1, D), lambda i: (0, 0)),
        ]
    else:
        in_specs = [
            pl.BlockSpec((2, BLK, D), lambda i: (0, i, 0)),
            pl.BlockSpec((NW, BLK), lambda i: (0, i)),
            pl.BlockSpec((1, D), lambda i: (0, 0)),
            pl.BlockSpec((D, D), lambda i: (0, 0)),
            pl.BlockSpec((1, D), lambda i: (0, 0)),
            pl.BlockSpec((1, D), lambda i: (0, 0)),
        ]
    out_specs = [
        pl.BlockSpec((BLK, D), lambda i: (i, 0)),
        pl.BlockSpec((8, BLK), lambda i: (0, i)),
    ]
    return pl.pallas_call(
        functools.partial(_tc_layer_body, first),
        grid=grid,
        in_specs=in_specs,
        out_specs=out_specs,
        out_shape=[
            jax.ShapeDtypeStruct((NP, D), jnp.float32),
            jax.ShapeDtypeStruct((8, NP), jnp.float32),
        ],
    )(t_or_p, s_parts, b_prev, W, a_src, a_dst)


NSUP = 40
SC_CH = 8
NCH = CH
NT = 10112
SR = 80
SCL = 128


def _sc_edge_body(h_hbm, ssrc_h, sdst_h, src_hbm, dst_hbm, z_hbm,
                  outp_hbm, sp_hbm,
                  ssrc_t, sdst_t, src_t, dst_t, dv_buf, e_buf, rows, s_loc,
                  out_acc,
                  gs0, gs1, ss0, ss1, xs, xd):
    cid = lax.axis_index("c")
    sid = lax.axis_index("s")
    wid = cid * NS + sid
    gs = (gs0, gs1)
    ssm = (ss0, ss1)
    i32 = jnp.int32

    pltpu.sync_copy(ssrc_h.at[pl.ds(0, NT)], ssrc_t)
    pltpu.sync_copy(sdst_h.at[pl.ds(0, NT)], sdst_t)

    def _zs(i, _):
        s_loc[lax.shift_right_logical(i, 3),
              pl.ds(jnp.bitwise_and(i, 7) * 16, 16)] = jnp.zeros(
                  (16,), jnp.float32)
        return 0
    lax.fori_loop(0, NP // 16, _zs, 0)

    pltpu.sync_copy(z_hbm.at[pl.ds(sid * ROWS_PER_TILE, ROWS_PER_TILE)],
                    out_acc.at[pl.ds(sid * ROWS_PER_TILE, ROWS_PER_TILE)])
    plsc.subcore_barrier()

    pltpu.sync_copy(src_hbm.at[wid, pl.ds(0, SC_CH)], src_t.at[0])
    pltpu.sync_copy(dst_hbm.at[wid, pl.ds(0, SC_CH)], dst_t.at[0])
    pltpu.async_copy(h_hbm.at[src_t.at[0, 0]], rows.at[0], gs0)

    def _wait(src_ref, dst_ref, sem):
        pltpu.make_async_copy(src_ref, dst_ref, sem).wait()

    def _iter(g, b):
        sup = lax.shift_right_logical(g, 3)
        slot = jnp.bitwise_and(sup, 1)
        row = jnp.bitwise_and(g, 7)

        @pl.when(jnp.logical_and(row == 0, sup + 1 < NSUP))
        def _():
            nslot = jnp.bitwise_and(sup + 1, 1)
            pltpu.async_copy(src_hbm.at[wid, pl.ds((sup + 1) * SC_CH, SC_CH)],
                             src_t.at[nslot], xs)
            pltpu.async_copy(dst_hbm.at[wid, pl.ds((sup + 1) * SC_CH, SC_CH)],
                             dst_t.at[nslot], xd)

        for gi in range(K // 16):
            sv = src_t[slot, row, pl.ds(gi * 16, 16)]
            dv = dst_t[slot, row, pl.ds(gi * 16, 16)]
            a1 = plsc.load_gather(ssrc_t, [sv])
            a2 = plsc.load_gather(sdst_t, [dv])
            al = a1 + a2
            al = jnp.where(al >= 0.0, al, al * 0.2)
            ev = jnp.exp(al)
            e_buf[pl.ds(gi * 16, 16)] = ev
            plsc.addupdate_scatter(
                s_loc, [lax.shift_right_logical(dv, 7),
                        jnp.bitwise_and(dv, 127)], ev)
            dv_buf[b, pl.ds(gi * 16, 16)] = dv

        b2 = 1 - b
        @pl.when(g + 1 < NCH)
        def _():
            g2 = g + 1
            sup2 = lax.shift_right_logical(g2, 3)
            slot2 = jnp.bitwise_and(sup2, 1)
            row2 = jnp.bitwise_and(g2, 7)

            @pl.when(row2 == 0)
            def _():
                _wait(src_hbm.at[wid, pl.ds(0, SC_CH)], src_t.at[slot2], xs)
                _wait(dst_hbm.at[wid, pl.ds(0, SC_CH)], dst_t.at[slot2], xd)

            @pl.when(g >= 1)
            def _():
                _wait(rows.at[b2], out_acc.at[pl.ds(0, K)], ssm[b2])

            pltpu.async_copy(h_hbm.at[src_t.at[slot2, row2]], rows.at[b2],
                             gs[b2])

        _wait(h_hbm.at[pl.ds(0, K)], rows.at[b], gs[b])

        def _scale(j2, _):
            j = j2 * 2
            for dj in range(2):
                es = plsc.load_gather(
                    e_buf, [jnp.full((16,), j + dj, i32)])
                for r in range(D // 16):
                    rows[b, j + dj, pl.ds(r * 16, 16)] = (
                        rows[b, j + dj, pl.ds(r * 16, 16)] * es)
            return 0
        lax.fori_loop(0, K // 2, _scale, 0)

        pltpu.async_copy(rows.at[b], out_acc.at[dv_buf.at[b]], ssm[b],
                         add=True)

    def _pair(i, _):
        _iter(i * 2, 0)
        _iter(i * 2 + 1, 1)
        return 0
    lax.fori_loop(0, NCH // 2, _pair, 0)

    pltpu.make_async_copy(rows.at[0], out_acc.at[pl.ds(0, K)], ss0).wait()
    pltpu.make_async_copy(rows.at[1], out_acc.at[pl.ds(0, K)], ss1).wait()
    plsc.subcore_barrier()

    pltpu.sync_copy(out_acc.at[pl.ds(sid * ROWS_PER_TILE, ROWS_PER_TILE)],
                    outp_hbm.at[cid, pl.ds(sid * ROWS_PER_TILE, ROWS_PER_TILE)])
    pltpu.sync_copy(s_loc, sp_hbm.at[wid])


def _sc_edges(h, ssrc_h, sdst_h, src_r, dst_r, zeros_hbm):
    mesh = plsc.VectorSubcoreMesh(core_axis_name="c", subcore_axis_name="s")
    f = pl.kernel(
        _sc_edge_body,
        out_type=[
            jax.ShapeDtypeStruct((NC, NP, D), jnp.float32),
            jax.ShapeDtypeStruct((NW, SR, SCL), jnp.float32),
        ],
        mesh=mesh,
        compiler_params=pltpu.CompilerParams(needs_layout_passes=False),
        scratch_types=[
            pltpu.VMEM((NT,), jnp.float32),
            pltpu.VMEM((NT,), jnp.float32),
            pltpu.VMEM((2, SC_CH, K), jnp.int32),
            pltpu.VMEM((2, SC_CH, K), jnp.int32),
            pltpu.VMEM((2, K), jnp.int32),
            pltpu.VMEM((K,), jnp.float32),
            pltpu.VMEM((2, K, D), jnp.float32),
            pltpu.VMEM((SR, SCL), jnp.float32),
            pltpu.VMEM_SHARED((NP, D), jnp.float32),
            pltpu.SemaphoreType.DMA,
            pltpu.SemaphoreType.DMA,
            pltpu.SemaphoreType.DMA,
            pltpu.SemaphoreType.DMA,
            pltpu.SemaphoreType.DMA,
            pltpu.SemaphoreType.DMA,
        ],
    )
    return f(h, ssrc_h, sdst_h, src_r, dst_r, zeros_hbm)


def _tc_head_body(p_ref, s_ref, b4_ref, batch_ref, energy_ref,
                  l0w_ref, l0b_ref, l1wa_ref, l1wb_ref, l1b_ref,
                  l2w_ref, l2b_ref, l3w_ref, l3b_ref, out_ref):
    agg = p_ref[0] + p_ref[1]
    s = jnp.sum(s_ref[...], axis=0)
    t = jnp.maximum(agg / (s[:, None] + 1e-16) + b4_ref[0][None, :], 0.0)

    bt = batch_ref[...].reshape(1, NP)
    classes = lax.broadcasted_iota(jnp.int32, (B, NP), 0)
    oh = (bt == classes).astype(jnp.float32)
    g = jnp.dot(oh, t, preferred_element_type=jnp.float32,
                precision=lax.Precision.HIGHEST)

    y = jnp.maximum(
        jnp.dot(g, l0w_ref[...], preferred_element_type=jnp.float32)
        + l0b_ref[0][None, :], 0.0)
    z = jnp.maximum(
        jnp.dot(y, l1wa_ref[...], preferred_element_type=jnp.float32)
        + energy_ref[...] * l1wb_ref[0][None, :] + l1b_ref[0][None, :], 0.0)
    z = z + y
    z = jnp.maximum(
        jnp.dot(z, l2w_ref[...], preferred_element_type=jnp.float32)
        + l2b_ref[0][None, :], 0.0)
    o = (jnp.dot(z, l3w_ref[...], preferred_element_type=jnp.float32)
         + l3b_ref[0][None, :])
    m = jnp.max(o, axis=1, keepdims=True)
    lse = m + jnp.log(jnp.sum(jnp.exp(o - m), axis=1, keepdims=True))
    out_ref[...] = o - lse


def _tc_head(p, s_parts, b4, batchs2d, energy,
             l0w, l0b, l1wa, l1wb, l1b, l2w, l2b, l3w, l3b):
    return pl.pallas_call(
        _tc_head_body,
        out_shape=jax.ShapeDtypeStruct((B, 2), jnp.float32),
    )(p, s_parts, b4, batchs2d, energy,
      l0w, l0b, l1wa, l1wb, l1b, l2w, l2b, l3w, l3b)


def kernel(x, edge_index, batchs, energy, conv1_W, conv1_asrc, conv1_adst,
           conv1_b, convs_W, convs_asrc, convs_adst, convs_b, lin0_W, lin0_b,
           lin1_W, lin1_b, lin2_W, lin2_b, lin3_W, lin3_b):
    f32 = jnp.float32
    x_pad = jnp.zeros((NP, D), f32).at[:N].set(x)
    src = jnp.concatenate(
        [edge_index[0], jnp.full((EP - E,), SENT, jnp.int32)]).reshape(NW, CH, K)
    dst = jnp.concatenate(
        [edge_index[1], jnp.zeros((EP - E,), jnp.int32)]).reshape(NW, CH, K)
    batchs2d = jnp.concatenate(
        [batchs, jnp.full((NP - N,), B, jnp.int32)]).reshape(NP // 128, 128)
    zeros_hbm = jnp.zeros((NP, D), f32)

    r2 = lambda v: v.reshape(1, D)

    h, ss = _tc_layer(True, x_pad, jnp.zeros((NW, NP), f32), r2(conv1_b),
                      conv1_W, r2(conv1_asrc), r2(conv1_adst))
    p, sp3 = _sc_edges(h, ss[0], ss[1], src, dst, zeros_hbm)
    sp = sp3.reshape(NW, NP)
    bias_prev = conv1_b

    for i in range(3):
        h, ss = _tc_layer(False, p, sp, r2(bias_prev), convs_W[i],
                          r2(convs_asrc[i]), r2(convs_adst[i]))
        p, sp3 = _sc_edges(h, ss[0], ss[1], src, dst, zeros_hbm)
        sp = sp3.reshape(NW, NP)
        bias_prev = convs_b[i]

    return _tc_head(p, sp, r2(bias_prev), batchs2d, energy,
                    lin0_W, lin0_b.reshape(1, D),
                    lin1_W[:D], lin1_W[D:D + 1], lin1_b.reshape(1, D),
                    lin2_W, lin2_b.reshape(1, D),
                    lin3_W, lin3_b.reshape(1, 2))
